# baseline (speedup 1.0000x reference)
import jax
import jax.numpy as jnp
from jax.experimental import pallas as pl
from jax.experimental.pallas import tpu as pltpu

_N = 250
_L = 128
_BM = 1000


def _clamp01(x):
    return jnp.minimum(jnp.maximum(x, 0.0), 1.0)


def _tel_kernel(x_ref, wh_ref, ws_ref, ar_ref, br_ref, out_ref, wf_ref):
    @pl.when(pl.program_id(0) == 0)
    def _prep():
        a = jax.nn.sigmoid(ar_ref[...])
        b = jax.nn.sigmoid(br_ref[...])

        wh = wh_ref[...]
        wh = jnp.exp(wh - jnp.max(wh, axis=0, keepdims=True))
        wh = wh * (a / jnp.sum(wh, axis=0, keepdims=True))

        ws = ws_ref[...]
        ws = jnp.exp(ws - jnp.max(ws, axis=0, keepdims=True))
        ws = ws * (b / jnp.sum(ws, axis=0, keepdims=True))

        c = 1.0 - _clamp01(a + b)
        wf_ref[...] = jnp.concatenate([wh, ws, c], axis=0)

    acc = jax.lax.dot_general(x_ref[...], wf_ref[...],
                              (((1,), (0,)), ((), ())),
                              preferred_element_type=jnp.float32)
    out_ref[...] = _clamp01(acc)


def kernel(input, input_all, all_states, t, entity2id, flag, w, w_inv,
           weight, alpha, beta):
    n = _N
    n_ent = entity2id.shape[0]
    m = input.shape[0]
    k = input.shape[1]

    w_t = w[t]
    wh_raw = w_t[:, :n].T
    ws_raw = w_t[:, n:].T
    a_raw = alpha[t, 0, :].reshape(1, _L)
    b_raw = beta[t, 0, :].reshape(1, _L)

    grid = (m // _BM,)
    out = pl.pallas_call(
        _tel_kernel,
        grid=grid,
        in_specs=[
            pl.BlockSpec((_BM, k), lambda i: (i, 0)),
            pl.BlockSpec((n, _L), lambda i: (0, 0)),
            pl.BlockSpec((n, _L), lambda i: (0, 0)),
            pl.BlockSpec((1, _L), lambda i: (0, 0)),
            pl.BlockSpec((1, _L), lambda i: (0, 0)),
        ],
        out_specs=pl.BlockSpec((_BM, _L), lambda i: (i, 0)),
        out_shape=jax.ShapeDtypeStruct((m, _L), jnp.float32),
        scratch_shapes=[pltpu.VMEM((2 * n + 1, _L), jnp.float32)],
    )(input, wh_raw, ws_raw, a_raw, b_raw)

    return out.reshape(n_ent, -1, _L)


# speedup vs baseline: 1.1054x; 1.1054x over previous
import jax
import jax.numpy as jnp
from jax.experimental import pallas as pl
from jax.experimental.pallas import tpu as pltpu

_N = 250
_L = 128
_BM = 2000


def _clamp01(x):
    return jnp.minimum(jnp.maximum(x, 0.0), 1.0)


def _tel_kernel(x_ref, wh_ref, ws_ref, ar_ref, br_ref, out_ref):
    a = jax.nn.sigmoid(ar_ref[...])
    b = jax.nn.sigmoid(br_ref[...])

    wh = wh_ref[...]
    wh = jnp.exp(wh - jnp.max(wh, axis=0, keepdims=True))
    wh = wh * (a / jnp.sum(wh, axis=0, keepdims=True))

    ws = ws_ref[...]
    ws = jnp.exp(ws - jnp.max(ws, axis=0, keepdims=True))
    ws = ws * (b / jnp.sum(ws, axis=0, keepdims=True))

    c = 1.0 - _clamp01(a + b)

    w_full = jnp.concatenate([wh, ws, c], axis=0)

    acc = jax.lax.dot_general(x_ref[...], w_full,
                              (((1,), (0,)), ((), ())),
                              preferred_element_type=jnp.float32)
    out_ref[...] = _clamp01(acc)


def kernel(input, input_all, all_states, t, entity2id, flag, w, w_inv,
           weight, alpha, beta):
    n = _N
    n_ent = entity2id.shape[0]
    m = input.shape[0]
    k = input.shape[1]

    w_t = w[t]
    wh_raw = w_t[:, :n].T
    ws_raw = w_t[:, n:].T
    a_raw = alpha[t, 0, :].reshape(1, _L)
    b_raw = beta[t, 0, :].reshape(1, _L)

    grid = (m // _BM,)
    out = pl.pallas_call(
        _tel_kernel,
        grid=grid,
        in_specs=[
            pl.BlockSpec((_BM, k), lambda i: (i, 0)),
            pl.BlockSpec((n, _L), lambda i: (0, 0)),
            pl.BlockSpec((n, _L), lambda i: (0, 0)),
            pl.BlockSpec((1, _L), lambda i: (0, 0)),
            pl.BlockSpec((1, _L), lambda i: (0, 0)),
        ],
        out_specs=pl.BlockSpec((_BM, _L), lambda i: (i, 0)),
        out_shape=jax.ShapeDtypeStruct((m, _L), jnp.float32),
        compiler_params=pltpu.CompilerParams(
            dimension_semantics=("parallel",)),
    )(input, wh_raw, ws_raw, a_raw, b_raw)

    return out.reshape(n_ent, -1, _L)
